# Initial kernel scaffold; baseline (speedup 1.0000x reference)
import jax
import jax.numpy as jnp
from jax.experimental import pallas as pl


def kernel(embed, h, c, edge_index, W_f, bw_f, b_f, W_i, bw_i, b_i, W_u, bw_u, b_u, W_o, bw_o, b_o):
    raise NotImplementedError("write your pallas kernel here")


# speedup vs baseline: 2.9804x; 2.9804x over previous
import functools

import jax
import jax.numpy as jnp
from jax import lax
from jax.experimental import pallas as pl
from jax.experimental.pallas import tpu as pltpu
from jax.experimental.pallas import tpu_sc as plsc

N = 10000
E = 320000
H = 128

NC = 2
NS = 16
CHW = 128
EP = 327680
TPE = EP // (NC * NS)
NCHK = TPE // CHW
ACC_N = 10240
STRIPE = ACC_N // NS

_mesh = plsc.VectorSubcoreMesh(
    core_axis_name="c", subcore_axis_name="s", num_cores=NC, num_subcores=NS)

_part = jax.ShapeDtypeStruct((NC * ACC_N, H), jnp.float32)


@functools.partial(
    pl.kernel,
    out_type=[_part, _part, _part],
    mesh=_mesh,
    scratch_types=[
        pltpu.VMEM((NCHK, CHW), jnp.int32),
        pltpu.VMEM((NCHK, CHW), jnp.int32),
        pltpu.VMEM((CHW, H), jnp.float32),
        pltpu.VMEM_SHARED((ACC_N, H), jnp.float32),
        pltpu.SemaphoreType.DMA,
    ],
)
def _segsum_kernel(src_hbm, dst_hbm, h_hbm, x_hbm, c_hbm,
                   outh_hbm, outx_hbm, outc_hbm,
                   srcb, dstb, rows, acc, sem):
    c = lax.axis_index("c")
    s = lax.axis_index("s")
    w = c * NS + s

    pltpu.sync_copy(src_hbm.at[pl.ds(w * NCHK, NCHK)], srcb)
    pltpu.sync_copy(dst_hbm.at[pl.ds(w * NCHK, NCHK)], dstb)

    for feat_hbm, out_hbm in ((h_hbm, outh_hbm), (x_hbm, outx_hbm),
                              (c_hbm, outc_hbm)):
        zv = jnp.zeros((16,), jnp.float32)

        def _zrow(r, _):
            def _zcol(k, _):
                rows[r, pl.ds(k * 16, 16)] = zv
                return 0
            return lax.fori_loop(0, H // 16, _zcol, 0)
        lax.fori_loop(0, CHW, _zrow, 0)

        def _zacc(k, _):
            pltpu.sync_copy(rows, acc.at[pl.ds(s * STRIPE + k * CHW, CHW)])
            return 0
        lax.fori_loop(0, STRIPE // CHW, _zacc, 0)
        plsc.subcore_barrier()

        def _chunk(j, _):
            pltpu.async_copy(feat_hbm.at[srcb.at[j]], rows, sem).wait()
            pltpu.sync_copy(rows, acc.at[dstb.at[j]], add=True)
            return 0
        lax.fori_loop(0, NCHK, _chunk, 0)
        plsc.subcore_barrier()

        pltpu.sync_copy(acc.at[pl.ds(s * STRIPE, STRIPE)],
                        out_hbm.at[pl.ds(c * ACC_N + s * STRIPE, STRIPE)])


def _gates_body(hp_ref, xp_ref, cp_ref, wh_ref, wx_ref, b_ref, h_ref, c_ref):
    hsum = hp_ref[0] + hp_ref[1]
    xsum = xp_ref[0] + xp_ref[1]
    csum = cp_ref[0] + cp_ref[1]
    gp = (jnp.dot(hsum, wh_ref[:], preferred_element_type=jnp.float32)
          + jnp.dot(xsum, wx_ref[:], preferred_element_type=jnp.float32)
          + b_ref[:])
    f = jax.nn.sigmoid(gp[:, :H])
    i = jax.nn.sigmoid(gp[:, H:2 * H])
    u = jnp.tanh(gp[:, 2 * H:3 * H])
    o = jax.nn.sigmoid(gp[:, 3 * H:])
    c_new = i * u + f * csum
    h_ref[:] = o * jnp.tanh(c_new)
    c_ref[:] = c_new


def _gates(hp, xp, cp, wh, wx, bias):
    blk = 256
    grid = (ACC_N // blk,)
    part_spec = pl.BlockSpec((NC, blk, H), lambda i: (0, i, 0))
    return pl.pallas_call(
        _gates_body,
        grid=grid,
        in_specs=[
            part_spec, part_spec, part_spec,
            pl.BlockSpec((H, 4 * H), lambda i: (0, 0)),
            pl.BlockSpec((H, 4 * H), lambda i: (0, 0)),
            pl.BlockSpec((1, 4 * H), lambda i: (0, 0)),
        ],
        out_specs=[
            pl.BlockSpec((blk, H), lambda i: (i, 0)),
            pl.BlockSpec((blk, H), lambda i: (i, 0)),
        ],
        out_shape=[
            jax.ShapeDtypeStruct((ACC_N, H), jnp.float32),
            jax.ShapeDtypeStruct((ACC_N, H), jnp.float32),
        ],
    )(hp, xp, cp, wh, wx, bias)


def kernel(embed, h, c, edge_index, W_f, bw_f, b_f, W_i, bw_i, b_i,
           W_u, bw_u, b_u, W_o, bw_o, b_o):
    src = jnp.concatenate(
        [edge_index[0].astype(jnp.int32),
         jnp.zeros((EP - E,), jnp.int32)]).reshape(EP // CHW, CHW)
    dst = jnp.concatenate(
        [edge_index[1].astype(jnp.int32),
         jnp.full((EP - E,), N, jnp.int32)]).reshape(EP // CHW, CHW)
    sh, sx, sc = _segsum_kernel(src, dst, h, embed, c)
    hp = sh.reshape(NC, ACC_N, H)
    xp = sx.reshape(NC, ACC_N, H)
    cp = sc.reshape(NC, ACC_N, H)
    w_cat = jnp.concatenate([W_f.T, W_i.T, W_u.T, W_o.T], axis=1)
    bias = jnp.concatenate([bw_f + b_f, bw_i + b_i,
                            bw_u + b_u, bw_o + b_o]).reshape(1, 4 * H)
    h_new, c_new = _gates(hp, xp, cp, w_cat[:H], w_cat[H:], bias)
    return h_new[:N], c_new[:N]


# speedup vs baseline: 3.3574x; 1.1265x over previous
import functools

import jax
import jax.numpy as jnp
from jax import lax
from jax.experimental import pallas as pl
from jax.experimental.pallas import tpu as pltpu
from jax.experimental.pallas import tpu_sc as plsc

N = 10000
E = 320000
H = 128

NC = 2
NS = 16
CHW = 128
EP = 327680
TPE = EP // (NC * NS)
NCHK = TPE // CHW
HCHK = NCHK // 2
ACC_N = 10240
STRIPE = ACC_N // NS

_mesh = plsc.VectorSubcoreMesh(
    core_axis_name="c", subcore_axis_name="s", num_cores=NC, num_subcores=NS)

_part = jax.ShapeDtypeStruct((NC * ACC_N, H), jnp.float32)


@functools.partial(
    pl.kernel,
    out_type=[_part, _part, _part],
    mesh=_mesh,
    scratch_types=[
        pltpu.VMEM((HCHK, CHW), jnp.int32),
        pltpu.VMEM((HCHK, CHW), jnp.int32),
        pltpu.VMEM((CHW, H), jnp.float32),
        pltpu.VMEM((CHW, H), jnp.float32),
        pltpu.VMEM_SHARED((ACC_N, H), jnp.float32),
        pltpu.SemaphoreType.DMA,
        pltpu.SemaphoreType.DMA,
    ],
)
def _segsum_kernel(src_hbm, dst_hbm, h_hbm, x_hbm, c_hbm,
                   outh_hbm, outx_hbm, outc_hbm,
                   srcb, dstb, rows, rows1, acc, sem, sem1):
    c = lax.axis_index("c")
    s = lax.axis_index("s")
    w = c * NS + s

    for feat_hbm, out_hbm in ((h_hbm, outh_hbm), (x_hbm, outx_hbm),
                              (c_hbm, outc_hbm)):
        zv = jnp.zeros((16,), jnp.float32)

        def _zrow(r, _):
            def _zcol(k, _):
                rows[r, pl.ds(k * 16, 16)] = zv
                return 0
            return lax.fori_loop(0, H // 16, _zcol, 0)
        lax.fori_loop(0, CHW, _zrow, 0)

        def _zacc(k, _):
            pltpu.sync_copy(rows, acc.at[pl.ds(s * STRIPE + k * CHW, CHW)])
            return 0
        lax.fori_loop(0, STRIPE // CHW, _zacc, 0)
        plsc.subcore_barrier()

        for half in range(2):
            pltpu.sync_copy(src_hbm.at[pl.ds(w * NCHK + half * HCHK, HCHK)],
                            srcb)
            pltpu.sync_copy(dst_hbm.at[pl.ds(w * NCHK + half * HCHK, HCHK)],
                            dstb)
            pltpu.async_copy(feat_hbm.at[srcb.at[0]], rows, sem)

            def _pair(jj, _):
                j0 = 2 * jj
                j1 = j0 + 1
                pltpu.async_copy(feat_hbm.at[srcb.at[j1]], rows1, sem1)
                pltpu.make_async_copy(feat_hbm.at[srcb.at[j0]], rows,
                                      sem).wait()
                pltpu.sync_copy(rows, acc.at[dstb.at[j0]], add=True)

                @pl.when(j1 + 1 < HCHK)
                def _():
                    pltpu.async_copy(feat_hbm.at[srcb.at[j1 + 1]], rows, sem)
                pltpu.make_async_copy(feat_hbm.at[srcb.at[j1]], rows1,
                                      sem1).wait()
                pltpu.sync_copy(rows1, acc.at[dstb.at[j1]], add=True)
                return 0
            lax.fori_loop(0, HCHK // 2, _pair, 0)
        plsc.subcore_barrier()

        pltpu.sync_copy(acc.at[pl.ds(s * STRIPE, STRIPE)],
                        out_hbm.at[pl.ds(c * ACC_N + s * STRIPE, STRIPE)])


def _gates_body(hp_ref, xp_ref, cp_ref, wh_ref, wx_ref, b_ref, h_ref, c_ref):
    hsum = hp_ref[0] + hp_ref[1]
    xsum = xp_ref[0] + xp_ref[1]
    csum = cp_ref[0] + cp_ref[1]
    gp = (jnp.dot(hsum, wh_ref[:], preferred_element_type=jnp.float32)
          + jnp.dot(xsum, wx_ref[:], preferred_element_type=jnp.float32)
          + b_ref[:])
    f = jax.nn.sigmoid(gp[:, :H])
    i = jax.nn.sigmoid(gp[:, H:2 * H])
    u = jnp.tanh(gp[:, 2 * H:3 * H])
    o = jax.nn.sigmoid(gp[:, 3 * H:])
    c_new = i * u + f * csum
    h_ref[:] = o * jnp.tanh(c_new)
    c_ref[:] = c_new


def _gates(hp, xp, cp, wh, wx, bias):
    blk = 256
    grid = (ACC_N // blk,)
    part_spec = pl.BlockSpec((NC, blk, H), lambda i: (0, i, 0))
    return pl.pallas_call(
        _gates_body,
        grid=grid,
        in_specs=[
            part_spec, part_spec, part_spec,
            pl.BlockSpec((H, 4 * H), lambda i: (0, 0)),
            pl.BlockSpec((H, 4 * H), lambda i: (0, 0)),
            pl.BlockSpec((1, 4 * H), lambda i: (0, 0)),
        ],
        out_specs=[
            pl.BlockSpec((blk, H), lambda i: (i, 0)),
            pl.BlockSpec((blk, H), lambda i: (i, 0)),
        ],
        out_shape=[
            jax.ShapeDtypeStruct((ACC_N, H), jnp.float32),
            jax.ShapeDtypeStruct((ACC_N, H), jnp.float32),
        ],
    )(hp, xp, cp, wh, wx, bias)


def kernel(embed, h, c, edge_index, W_f, bw_f, b_f, W_i, bw_i, b_i,
           W_u, bw_u, b_u, W_o, bw_o, b_o):
    src = jnp.concatenate(
        [edge_index[0].astype(jnp.int32),
         jnp.zeros((EP - E,), jnp.int32)]).reshape(EP // CHW, CHW)
    dst = jnp.concatenate(
        [edge_index[1].astype(jnp.int32),
         jnp.full((EP - E,), N, jnp.int32)]).reshape(EP // CHW, CHW)
    sh, sx, sc = _segsum_kernel(src, dst, h, embed, c)
    hp = sh.reshape(NC, ACC_N, H)
    xp = sx.reshape(NC, ACC_N, H)
    cp = sc.reshape(NC, ACC_N, H)
    w_cat = jnp.concatenate([W_f.T, W_i.T, W_u.T, W_o.T], axis=1)
    bias = jnp.concatenate([bw_f + b_f, bw_i + b_i,
                            bw_u + b_u, bw_o + b_o]).reshape(1, 4 * H)
    h_new, c_new = _gates(hp, xp, cp, w_cat[:H], w_cat[H:], bias)
    return h_new[:N], c_new[:N]


# speedup vs baseline: 3.6819x; 1.0967x over previous
import functools

import jax
import jax.numpy as jnp
from jax import lax
from jax.experimental import pallas as pl
from jax.experimental.pallas import tpu as pltpu
from jax.experimental.pallas import tpu_sc as plsc

N = 10000
E = 320000
H = 128

NC = 2
NS = 16
CHW = 128
EP = 327680
Q0 = 128
Q1 = 32
HCHK = 32
MAXG = Q0 // HCHK
ACC_N = 10240
STRIPE = ACC_N // NS

_mesh = plsc.VectorSubcoreMesh(
    core_axis_name="c", subcore_axis_name="s", num_cores=NC, num_subcores=NS)

_part = jax.ShapeDtypeStruct((NC * ACC_N, H), jnp.float32)


@functools.partial(
    pl.kernel,
    out_type=[_part, _part, _part],
    mesh=_mesh,
    scratch_types=[
        pltpu.VMEM((HCHK, CHW), jnp.int32),
        pltpu.VMEM((HCHK, CHW), jnp.int32),
        pltpu.VMEM((CHW, H), jnp.float32),
        pltpu.VMEM((CHW, H), jnp.float32),
        pltpu.VMEM_SHARED((ACC_N, H), jnp.float32),
        pltpu.SemaphoreType.DMA,
        pltpu.SemaphoreType.DMA,
    ],
)
def _segsum_kernel(src_hbm, dst_hbm, h_hbm, x_hbm, c_hbm,
                   outh_hbm, outx_hbm, outc_hbm,
                   srcb, dstb, rows, rows1, acc, sem, sem1):
    c = lax.axis_index("c")
    s = lax.axis_index("s")
    q = jnp.where(c == 0, Q0, Q1)
    cbase = c * (NS * Q0) + s * q
    ngrp = q // HCHK

    for feat_hbm, out_hbm in ((h_hbm, outh_hbm), (x_hbm, outx_hbm),
                              (c_hbm, outc_hbm)):
        zv = jnp.zeros((16,), jnp.float32)

        def _zrow(r, _):
            def _zcol(k, _):
                rows[r, pl.ds(k * 16, 16)] = zv
                return 0
            return lax.fori_loop(0, H // 16, _zcol, 0)
        lax.fori_loop(0, CHW, _zrow, 0)

        def _zacc(k, _):
            pltpu.sync_copy(rows, acc.at[pl.ds(s * STRIPE + k * CHW, CHW)])
            return 0
        lax.fori_loop(0, STRIPE // CHW, _zacc, 0)
        plsc.subcore_barrier()

        for grp in range(MAXG):
            @pl.when(grp < ngrp)
            def _():
                gb = cbase + grp * HCHK
                pltpu.sync_copy(src_hbm.at[pl.ds(gb, HCHK)], srcb)
                pltpu.sync_copy(dst_hbm.at[pl.ds(gb, HCHK)], dstb)
                pltpu.async_copy(feat_hbm.at[srcb.at[0]], rows, sem)

                def _pair(jj, _):
                    j0 = 2 * jj
                    j1 = j0 + 1
                    pltpu.async_copy(feat_hbm.at[srcb.at[j1]], rows1, sem1)
                    pltpu.make_async_copy(feat_hbm.at[srcb.at[j0]], rows,
                                          sem).wait()
                    pltpu.sync_copy(rows, acc.at[dstb.at[j0]], add=True)

                    @pl.when(j1 + 1 < HCHK)
                    def _():
                        pltpu.async_copy(feat_hbm.at[srcb.at[j1 + 1]], rows,
                                         sem)
                    pltpu.make_async_copy(feat_hbm.at[srcb.at[j1]], rows1,
                                          sem1).wait()
                    pltpu.sync_copy(rows1, acc.at[dstb.at[j1]], add=True)
                    return 0
                lax.fori_loop(0, HCHK // 2, _pair, 0)
        plsc.subcore_barrier()

        pltpu.sync_copy(acc.at[pl.ds(s * STRIPE, STRIPE)],
                        out_hbm.at[pl.ds(c * ACC_N + s * STRIPE, STRIPE)])


def _gates_body(hp_ref, xp_ref, cp_ref, wh_ref, wx_ref, b_ref, h_ref, c_ref):
    hsum = hp_ref[0] + hp_ref[1]
    xsum = xp_ref[0] + xp_ref[1]
    csum = cp_ref[0] + cp_ref[1]
    gp = (jnp.dot(hsum, wh_ref[:], preferred_element_type=jnp.float32)
          + jnp.dot(xsum, wx_ref[:], preferred_element_type=jnp.float32)
          + b_ref[:])
    f = jax.nn.sigmoid(gp[:, :H])
    i = jax.nn.sigmoid(gp[:, H:2 * H])
    u = jnp.tanh(gp[:, 2 * H:3 * H])
    o = jax.nn.sigmoid(gp[:, 3 * H:])
    c_new = i * u + f * csum
    h_ref[:] = o * jnp.tanh(c_new)
    c_ref[:] = c_new


def _gates(hp, xp, cp, wh, wx, bias):
    blk = 256
    grid = (ACC_N // blk,)
    part_spec = pl.BlockSpec((NC, blk, H), lambda i: (0, i, 0))
    return pl.pallas_call(
        _gates_body,
        grid=grid,
        in_specs=[
            part_spec, part_spec, part_spec,
            pl.BlockSpec((H, 4 * H), lambda i: (0, 0)),
            pl.BlockSpec((H, 4 * H), lambda i: (0, 0)),
            pl.BlockSpec((1, 4 * H), lambda i: (0, 0)),
        ],
        out_specs=[
            pl.BlockSpec((blk, H), lambda i: (i, 0)),
            pl.BlockSpec((blk, H), lambda i: (i, 0)),
        ],
        out_shape=[
            jax.ShapeDtypeStruct((ACC_N, H), jnp.float32),
            jax.ShapeDtypeStruct((ACC_N, H), jnp.float32),
        ],
    )(hp, xp, cp, wh, wx, bias)


def kernel(embed, h, c, edge_index, W_f, bw_f, b_f, W_i, bw_i, b_i,
           W_u, bw_u, b_u, W_o, bw_o, b_o):
    src = jnp.concatenate(
        [edge_index[0].astype(jnp.int32),
         jnp.zeros((EP - E,), jnp.int32)]).reshape(EP // CHW, CHW)
    dst = jnp.concatenate(
        [edge_index[1].astype(jnp.int32),
         jnp.full((EP - E,), N, jnp.int32)]).reshape(EP // CHW, CHW)
    sh, sx, sc = _segsum_kernel(src, dst, h, embed, c)
    hp = sh.reshape(NC, ACC_N, H)
    xp = sx.reshape(NC, ACC_N, H)
    cp = sc.reshape(NC, ACC_N, H)
    w_cat = jnp.concatenate([W_f.T, W_i.T, W_u.T, W_o.T], axis=1)
    bias = jnp.concatenate([bw_f + b_f, bw_i + b_i,
                            bw_u + b_u, bw_o + b_o]).reshape(1, 4 * H)
    h_new, c_new = _gates(hp, xp, cp, w_cat[:H], w_cat[H:], bias)
    return h_new[:N], c_new[:N]
